# baseline (speedup 1.0000x reference)
import functools

import jax
import jax.numpy as jnp
from jax.experimental import pallas as pl
from jax.experimental.pallas import tpu as pltpu


def _khop_body(a_ref, x_ref, wk_ref, b_ref, out_ref, s_ref):
    i = pl.program_id(1)
    cb = pl.program_id(2)
    ncb = pl.num_programs(2)
    part = jnp.dot(a_ref[0], x_ref[...], preferred_element_type=jnp.float32)

    @pl.when(cb == 0)
    def _():
        s_ref[...] = part

    @pl.when(cb > 0)
    def _():
        s_ref[...] += part

    @pl.when(cb == ncb - 1)
    def _():
        contrib = jnp.dot(s_ref[...], wk_ref[0], preferred_element_type=jnp.float32)

        @pl.when(i == 0)
        def _():
            out_ref[...] = contrib + b_ref[...]

        @pl.when(i > 0)
        def _():
            out_ref[...] += contrib


@functools.partial(jax.jit, static_argnames=("block_rows", "block_cols"))
def _khop(x, adj_list, wk, b2, block_rows, block_cols):
    k, n, _ = adj_list.shape
    d_in = x.shape[1]
    d_out = wk.shape[2]
    grid = (n // block_rows, k, n // block_cols)
    return pl.pallas_call(
        _khop_body,
        grid=grid,
        in_specs=[
            pl.BlockSpec((1, block_rows, block_cols), lambda rb, i, cb: (i, rb, cb)),
            pl.BlockSpec((block_cols, d_in), lambda rb, i, cb: (cb, 0)),
            pl.BlockSpec((1, d_in, d_out), lambda rb, i, cb: (i, 0, 0)),
            pl.BlockSpec((1, d_out), lambda rb, i, cb: (0, 0)),
        ],
        out_specs=pl.BlockSpec((block_rows, d_out), lambda rb, i, cb: (rb, 0)),
        out_shape=jax.ShapeDtypeStruct((n, d_out), jnp.float32),
        scratch_shapes=[pltpu.VMEM((block_rows, d_in), jnp.float32)],
        compiler_params=pltpu.CompilerParams(
            dimension_semantics=("parallel", "arbitrary", "arbitrary"),
            vmem_limit_bytes=100 * 1024 * 1024,
        ),
    )(adj_list, x, wk, b2)


def kernel(x, adj_list, W, b):
    k, n, _ = adj_list.shape
    d_in = x.shape[1]
    d_out = W.shape[0]
    wk = W.reshape(d_out, k, d_in).transpose(1, 2, 0)
    b2 = b.reshape(1, d_out)
    return _khop(x, adj_list, wk, b2, block_rows=512, block_cols=2048)


# speedup vs baseline: 1.3370x; 1.3370x over previous
import functools

import jax
import jax.numpy as jnp
from jax.experimental import pallas as pl
from jax.experimental.pallas import tpu as pltpu


def _khop_body(a_ref, x_ref, wk_ref, b_ref, out_ref, *, block_rows):
    i = pl.program_id(0)
    rb = pl.program_id(1)
    s = jnp.dot(a_ref[0], x_ref[...], preferred_element_type=jnp.float32)
    contrib = jnp.dot(s, wk_ref[0], preferred_element_type=jnp.float32)
    rows = pl.ds(rb * block_rows, block_rows)

    @pl.when(i == 0)
    def _():
        out_ref[rows, :] = contrib + b_ref[...]

    @pl.when(i > 0)
    def _():
        out_ref[rows, :] += contrib


@functools.partial(jax.jit, static_argnames=("block_rows",))
def _khop(x, adj_list, wk, b2, block_rows):
    k, n, _ = adj_list.shape
    d_in = x.shape[1]
    d_out = wk.shape[2]
    grid = (k, n // block_rows)
    return pl.pallas_call(
        functools.partial(_khop_body, block_rows=block_rows),
        grid=grid,
        in_specs=[
            pl.BlockSpec((1, block_rows, n), lambda i, rb: (i, rb, 0)),
            pl.BlockSpec((n, d_in), lambda i, rb: (0, 0)),
            pl.BlockSpec((1, d_in, d_out), lambda i, rb: (i, 0, 0)),
            pl.BlockSpec((1, d_out), lambda i, rb: (0, 0)),
        ],
        out_specs=pl.BlockSpec((n, d_out), lambda i, rb: (0, 0)),
        out_shape=jax.ShapeDtypeStruct((n, d_out), jnp.float32),
        compiler_params=pltpu.CompilerParams(
            dimension_semantics=("arbitrary", "arbitrary"),
            vmem_limit_bytes=100 * 1024 * 1024,
        ),
    )(adj_list, x, wk, b2)


def kernel(x, adj_list, W, b):
    k, n, _ = adj_list.shape
    d_in = x.shape[1]
    d_out = W.shape[0]
    wk = W.reshape(d_out, k, d_in).transpose(1, 2, 0)
    b2 = b.reshape(1, d_out)
    return _khop(x, adj_list, wk, b2, block_rows=512)
